# baseline (speedup 1.0000x reference)
import functools

import jax
import jax.numpy as jnp
from jax import lax
from jax.experimental import pallas as pl
from jax.experimental.pallas import tpu as pltpu
from jax.experimental.pallas import tpu_sc as plsc

NC = 2
NS = 16
L = 16
NW = NC * NS

N_PHOTONS = 8388608
PER_W = N_PHOTONS // NW
CHUNK = 16384
NCHUNK = PER_W // CHUNK
UNROLL = 8

N_PIXELS = 19
BINS_PAD = 24
HIST = BINS_PAD * L

RMAGIC = 12582912.0


@functools.lru_cache(maxsize=None)
def _sc_call(lut_rows, lut_cols):
    mesh = plsc.VectorSubcoreMesh(core_axis_name="c", subcore_axis_name="s")

    @functools.partial(
        pl.kernel,
        out_type=jax.ShapeDtypeStruct((NW, HIST), jnp.float32),
        mesh=mesh,
        compiler_params=pltpu.CompilerParams(needs_layout_passes=False),
        scratch_types=[
            pltpu.VMEM((CHUNK,), jnp.float32),
            pltpu.VMEM((CHUNK,), jnp.float32),
            pltpu.VMEM((CHUNK,), jnp.float32),
            pltpu.VMEM((CHUNK,), jnp.float32),
            pltpu.VMEM((CHUNK,), jnp.float32),
            pltpu.VMEM((CHUNK,), jnp.float32),
            pltpu.VMEM((64,), jnp.int32),
            pltpu.VMEM((16, L), jnp.float32),
            pltpu.VMEM((HIST,), jnp.float32),
            pltpu.SemaphoreType.DMA,
            pltpu.SemaphoreType.DMA,
        ],
    )
    def hexbin(x_hbm, y_hbm, v_hbm, tab_hbm, par_hbm, out_hbm,
               xb0, xb1, yb0, yb1, vb0, vb1, tab, par, hist, sem0, sem1):
        cid = lax.axis_index("c")
        sid = lax.axis_index("s")
        wid = sid * NC + cid
        base = wid * PER_W

        pltpu.sync_copy(tab_hbm, tab)
        pltpu.sync_copy(par_hbm, par)

        zero = jnp.zeros((L,), jnp.float32)
        for i in range(BINS_PAD):
            hist[pl.ds(i * L, L)] = zero

        a_v = par[0]
        b_v = par[1]
        c_v = par[2]
        d_v = par[3]
        e_v = par[4]
        rm = jnp.full((L,), RMAGIC, jnp.float32)
        lane = lax.iota(jnp.int32, L)
        sems = (sem0, sem1)
        bufs = ((xb0, yb0, vb0), (xb1, yb1, vb1))

        def start(g, slot):
            off = base + g * CHUNK
            xr, yr, vr = bufs[slot]
            return (
                pltpu.async_copy(x_hbm.at[pl.ds(off, CHUNK)], xr, sems[slot]),
                pltpu.async_copy(y_hbm.at[pl.ds(off, CHUNK)], yr, sems[slot]),
                pltpu.async_copy(v_hbm.at[pl.ds(off, CHUNK)], vr, sems[slot]),
            )

        wcols = float(lut_cols + 2)

        def compute(slot):
            xr, yr, vr = bufs[slot]

            def phase1(b0):
                pixs = []
                vvs = []
                for u in range(UNROLL):
                    sl = pl.ds(b0 + u * L, L)
                    xv = xr[sl]
                    yv = yr[sl]
                    vvs.append(vr[sl])
                    q = xv * a_v + yv * b_v + d_v
                    r = yv * c_v + e_v
                    t = q + r
                    qr = (q + rm) - rm
                    rr = (r + rm) - rm
                    tr = (t + rm) - rm
                    qd = jnp.abs(qr - q)
                    rd = jnp.abs(rr - r)
                    sd = jnp.abs(t - tr)
                    qr2 = jnp.where(qd > jnp.maximum(rd, sd), tr - rr, qr)
                    rr2 = jnp.where(rd > jnp.maximum(qd, sd), tr - qr, rr)
                    qc = jnp.minimum(jnp.maximum(qr2, 0.0), float(lut_rows + 1))
                    rc = jnp.minimum(jnp.maximum(rr2, 0.0), float(lut_cols + 1))
                    flat = (qc * wcols + rc).astype(jnp.int32)
                    pixs.append(plsc.load_gather(tab, [flat]) | lane)
                return tuple(pixs), tuple(vvs)

            def body(i, carry):
                cpix, cvv = carry
                npix, nvv = phase1(i * (L * UNROLL))
                for u in range(UNROLL):
                    plsc.addupdate_scatter(hist, [cpix[u]], cvv[u])
                return npix, nvv

            last = lax.fori_loop(1, CHUNK // (L * UNROLL), body, phase1(0))
            for u in range(UNROLL):
                plsc.addupdate_scatter(hist, [last[0][u]], last[1][u])

        pending = start(0, 0)
        for g in range(NCHUNK):
            slot = g & 1
            nxt = start(g + 1, 1 - slot) if g + 1 < NCHUNK else None
            for d in pending:
                d.wait()
            compute(slot)
            pending = nxt

        pltpu.sync_copy(hist, out_hbm.at[wid])

    return hexbin


def kernel(x, y, values, lookup_table, hex_size, q_offset, r_offset,
           q_min, r_min, n_pixels):
    lut_rows, lut_cols = lookup_table.shape
    h = jnp.float32(hex_size)
    s3 = jnp.sqrt(jnp.float32(3.0))
    qminf = jnp.float32(q_min)
    rminf = jnp.float32(r_min)
    par = jnp.stack([
        s3 / (3.0 * h),
        -1.0 / (3.0 * h),
        2.0 / (3.0 * h),
        -jnp.float32(q_offset) - (qminf - 1.0),
        -jnp.float32(r_offset) - (rminf - 1.0),
        *([jnp.float32(0.0)] * 11),
    ])
    par = jnp.broadcast_to(par[:, None], (16, L)).astype(jnp.float32)
    bordered = jnp.pad(lookup_table.astype(jnp.int32), 1, constant_values=-1)
    bordered = jnp.where(bordered < 0, N_PIXELS, bordered) * L
    flat_lut = bordered.reshape(-1)
    tab = jnp.concatenate(
        [flat_lut,
         jnp.full((64 - flat_lut.shape[0],), N_PIXELS * L, jnp.int32)])
    partials = _sc_call(lut_rows, lut_cols)(x, y, values, tab, par)
    return partials.reshape(NW, BINS_PAD, L).sum(axis=(0, 2))[:N_PIXELS]


# speedup vs baseline: 1.0887x; 1.0887x over previous
import functools

import jax
import jax.numpy as jnp
from jax import lax
from jax.experimental import pallas as pl
from jax.experimental.pallas import tpu as pltpu
from jax.experimental.pallas import tpu_sc as plsc

NC = 2
NS = 16
L = 16
NW = NC * NS

N_PHOTONS = 8388608
PER_W = N_PHOTONS // NW
CHUNK = 16384
NCHUNK = PER_W // CHUNK
UNROLL = 16

N_PIXELS = 19
BINS_PAD = 24
HIST = BINS_PAD * L

RMAGIC = 12582912.0


@functools.lru_cache(maxsize=None)
def _sc_call(lut_rows, lut_cols):
    mesh = plsc.VectorSubcoreMesh(core_axis_name="c", subcore_axis_name="s")

    @functools.partial(
        pl.kernel,
        out_type=jax.ShapeDtypeStruct((NW, HIST), jnp.float32),
        mesh=mesh,
        compiler_params=pltpu.CompilerParams(needs_layout_passes=False),
        scratch_types=[
            pltpu.VMEM((CHUNK,), jnp.float32),
            pltpu.VMEM((CHUNK,), jnp.float32),
            pltpu.VMEM((CHUNK,), jnp.float32),
            pltpu.VMEM((CHUNK,), jnp.float32),
            pltpu.VMEM((CHUNK,), jnp.float32),
            pltpu.VMEM((CHUNK,), jnp.float32),
            pltpu.VMEM((64,), jnp.int32),
            pltpu.VMEM((16, L), jnp.float32),
            pltpu.VMEM((HIST,), jnp.float32),
            pltpu.SemaphoreType.DMA,
            pltpu.SemaphoreType.DMA,
        ],
    )
    def hexbin(x_hbm, y_hbm, v_hbm, tab_hbm, par_hbm, out_hbm,
               xb0, xb1, yb0, yb1, vb0, vb1, tab, par, hist, sem0, sem1):
        cid = lax.axis_index("c")
        sid = lax.axis_index("s")
        wid = sid * NC + cid
        base = wid * PER_W

        pltpu.sync_copy(tab_hbm, tab)
        pltpu.sync_copy(par_hbm, par)

        zero = jnp.zeros((L,), jnp.float32)
        for i in range(BINS_PAD):
            hist[pl.ds(i * L, L)] = zero

        a_v = par[0]
        b_v = par[1]
        c_v = par[2]
        d_v = par[3]
        e_v = par[4]
        rm = jnp.full((L,), RMAGIC, jnp.float32)
        lane = lax.iota(jnp.int32, L)
        sems = (sem0, sem1)
        bufs = ((xb0, yb0, vb0), (xb1, yb1, vb1))

        def start(g, slot):
            off = base + g * CHUNK
            xr, yr, vr = bufs[slot]
            return (
                pltpu.async_copy(x_hbm.at[pl.ds(off, CHUNK)], xr, sems[slot]),
                pltpu.async_copy(y_hbm.at[pl.ds(off, CHUNK)], yr, sems[slot]),
                pltpu.async_copy(v_hbm.at[pl.ds(off, CHUNK)], vr, sems[slot]),
            )

        wcols = float(lut_cols + 2)

        def compute(slot):
            xr, yr, vr = bufs[slot]

            def phase1(b0):
                pixs = []
                vvs = []
                for u in range(UNROLL):
                    sl = pl.ds(b0 + u * L, L)
                    xv = xr[sl]
                    yv = yr[sl]
                    vvs.append(vr[sl])
                    q = xv * a_v + yv * b_v + d_v
                    r = yv * c_v + e_v
                    t = q + r
                    qr = (q + rm) - rm
                    rr = (r + rm) - rm
                    tr = (t + rm) - rm
                    qd = jnp.abs(qr - q)
                    rd = jnp.abs(rr - r)
                    sd = jnp.abs(t - tr)
                    qr2 = jnp.where(qd > jnp.maximum(rd, sd), tr - rr, qr)
                    rr2 = jnp.where(rd > jnp.maximum(qd, sd), tr - qr, rr)
                    qc = jnp.minimum(jnp.maximum(qr2, 0.0), float(lut_rows + 1))
                    rc = jnp.minimum(jnp.maximum(rr2, 0.0), float(lut_cols + 1))
                    flat = (qc * wcols + rc).astype(jnp.int32)
                    pixs.append(plsc.load_gather(tab, [flat]) | lane)
                for u in range(UNROLL):
                    plsc.addupdate_scatter(hist, [pixs[u]], vvs[u])

            def body(i, carry):
                phase1(i * (L * UNROLL))
                return carry

            lax.fori_loop(0, CHUNK // (L * UNROLL), body, 0)

        pending = start(0, 0)
        for g in range(NCHUNK):
            slot = g & 1
            nxt = start(g + 1, 1 - slot) if g + 1 < NCHUNK else None
            for d in pending:
                d.wait()
            compute(slot)
            pending = nxt

        pltpu.sync_copy(hist, out_hbm.at[wid])

    return hexbin


def kernel(x, y, values, lookup_table, hex_size, q_offset, r_offset,
           q_min, r_min, n_pixels):
    lut_rows, lut_cols = lookup_table.shape
    h = jnp.float32(hex_size)
    s3 = jnp.sqrt(jnp.float32(3.0))
    qminf = jnp.float32(q_min)
    rminf = jnp.float32(r_min)
    par = jnp.stack([
        s3 / (3.0 * h),
        -1.0 / (3.0 * h),
        2.0 / (3.0 * h),
        -jnp.float32(q_offset) - (qminf - 1.0),
        -jnp.float32(r_offset) - (rminf - 1.0),
        *([jnp.float32(0.0)] * 11),
    ])
    par = jnp.broadcast_to(par[:, None], (16, L)).astype(jnp.float32)
    bordered = jnp.pad(lookup_table.astype(jnp.int32), 1, constant_values=-1)
    bordered = jnp.where(bordered < 0, N_PIXELS, bordered) * L
    flat_lut = bordered.reshape(-1)
    tab = jnp.concatenate(
        [flat_lut,
         jnp.full((64 - flat_lut.shape[0],), N_PIXELS * L, jnp.int32)])
    partials = _sc_call(lut_rows, lut_cols)(x, y, values, tab, par)
    return partials.reshape(NW, BINS_PAD, L).sum(axis=(0, 2))[:N_PIXELS]
